# baseline (speedup 1.0000x reference)
import functools

import jax
import jax.numpy as jnp
from jax import lax
from jax.experimental import pallas as pl
from jax.experimental.pallas import tpu as pltpu
from jax.experimental.pallas import tpu_sc as plsc

_N_POINTS = 1000000
_N_DIM = 16
_NC = 2
_NS = 16
_NW = _NC * _NS
_LANE = 128
_NT_FULL = _N_POINTS // _LANE
_TAIL = _N_POINTS - _NT_FULL * _LANE
_CHT = 12
_CHW = _CHT * _LANE
_NCH = _NT_FULL // _CHT
_CPW = -(-_NCH // _NW)
_FULL_W = _NCH - _NW * (_CPW - 1)


def _sc_body(tt_hbm, ij_hbm, partials_hbm, rows_hbm,
             idx_v, tile_i, tile_j, rows_v, buf0, buf1, accb_v, acc_v,
             sem0, sem1, gsem):
  cid = lax.axis_index("c")
  sid = lax.axis_index("s")
  wid = sid * _NC + cid

  pltpu.sync_copy(ij_hbm, idx_v)
  idx = idx_v[...]
  ii = idx[0]
  jj = idx[1]
  base_i = pl.multiple_of((ii // _LANE) * _LANE, _LANE)
  base_j = pl.multiple_of((jj // _LANE) * _LANE, _LANE)
  pltpu.sync_copy(tt_hbm.at[:, pl.ds(base_i, _LANE)], tile_i)
  pltpu.sync_copy(tt_hbm.at[:, pl.ds(base_j, _LANE)], tile_j)
  col_i = jnp.full((_N_DIM,), ii % _LANE, jnp.int32)
  col_j = jnp.full((_N_DIM,), jj % _LANE, jnp.int32)
  dim_iota = lax.iota(jnp.int32, _N_DIM)
  tis = tuple(
      plsc.load_gather(tile_i, [jnp.full((_N_DIM,), d, jnp.int32), col_i])
      for d in range(_N_DIM))
  t2s = tuple(tis[d] + tis[d] for d in range(_N_DIM))
  rows_v[0, :] = plsc.load_gather(tile_i, [dim_iota, col_i])
  rows_v[1, :] = plsc.load_gather(tile_j, [dim_iota, col_j])

  bufs = (buf0, buf1)
  sems = (sem0, sem1)

  def start(c):
    g = c * _NW + wid
    if c == _CPW - 1:
      g = jnp.where(wid < _FULL_W, g, 0)
    off = pl.multiple_of(g * _CHW, _LANE)
    return pltpu.async_copy(
        tt_hbm.at[:, pl.ds(off, _CHW)], bufs[c % 2], sems[c % 2])

  zero_v = jnp.zeros((_N_DIM,), jnp.float32)
  for b in range(2 * _N_DIM):
    accb_v[b, :] = zero_v

  def chunk_sum(buf):
    @pl.loop(0, _CHW // (2 * _N_DIM))
    def _(l):
      for half in range(2):
        off = (l * 2 + half) * _N_DIM
        for d in range(_N_DIM):
          x = buf[d, pl.ds(off, _N_DIM)]
          e = jnp.exp(x * (t2s[d] - x))
          plsc.addupdate(accb_v.at[2 * d + half, :], e)

  inflight = start(0)
  for c in range(_CPW - 1):
    inflight.wait()
    nxt = start(c + 1)
    chunk_sum(bufs[c % 2])
    inflight = nxt
  inflight.wait()

  @pl.when(wid < _FULL_W)
  def _():
    chunk_sum(bufs[(_CPW - 1) % 2])

  for d in range(_N_DIM):
    k = jnp.exp(-(tis[d] * tis[d]))
    acc_v[d, :] = (accb_v[2 * d, :] + accb_v[2 * d + 1, :]) * k
  pltpu.sync_copy(acc_v, partials_hbm.at[wid])

  @pl.when(wid == 0)
  def _():
    pltpu.sync_copy(rows_v, rows_hbm)


@functools.partial(
    pl.kernel,
    out_type=(
        jax.ShapeDtypeStruct((_NW, _N_DIM, _N_DIM), jnp.float32),
        jax.ShapeDtypeStruct((2, _N_DIM), jnp.float32),
    ),
    mesh=plsc.VectorSubcoreMesh(core_axis_name="c", subcore_axis_name="s"),
    compiler_params=pltpu.CompilerParams(
        use_tc_tiling_on_sc=True, needs_layout_passes=False),
    scratch_types=(
        pltpu.VMEM((_N_DIM,), jnp.int32),
        pltpu.VMEM((_N_DIM, _LANE), jnp.float32),
        pltpu.VMEM((_N_DIM, _LANE), jnp.float32),
        pltpu.VMEM((2, _N_DIM), jnp.float32),
        pltpu.VMEM((_N_DIM, _CHW), jnp.float32),
        pltpu.VMEM((_N_DIM, _CHW), jnp.float32),
        pltpu.VMEM((2 * _N_DIM, _N_DIM), jnp.float32),
        pltpu.VMEM((_N_DIM, _N_DIM), jnp.float32),
        pltpu.SemaphoreType.DMA,
        pltpu.SemaphoreType.DMA,
        pltpu.SemaphoreType.DMA,
    ),
)
def _sc_reduce(tt_hbm, ij_hbm, partials_hbm, rows_hbm, *scratch):
  _sc_body(tt_hbm, ij_hbm, partials_hbm, rows_hbm, *scratch)


def _tc_epilogue(pij_ref, rows_ref, partials_ref, tail_ref, out_ref):
  ti = rows_ref[0:1, :]
  tj = rows_ref[1:2, :]
  q_sc = jnp.sum(partials_ref[...])
  dt = tail_ref[...] - ti
  q_tail = jnp.sum(jnp.exp(-(dt * dt)))
  q_sum = q_sc + q_tail - jnp.float32(_N_DIM)
  d = ti - tj
  p = pij_ref[...]
  t = p * (jnp.log(p) + d * d + jnp.log(q_sum))
  out_ref[...] = jnp.sum(t).reshape(1, 1)


def kernel(pij, i, j, table):
  ij = jnp.concatenate(
      [i.astype(jnp.int32), j.astype(jnp.int32),
       jnp.zeros((_N_DIM - 2,), jnp.int32)])
  tt = table.T
  partials, rows = _sc_reduce(tt, ij)
  tail = lax.slice(table, (_NT_FULL * _LANE, 0), (_N_POINTS, _N_DIM))
  out = pl.pallas_call(
      _tc_epilogue,
      out_shape=jax.ShapeDtypeStruct((1, 1), jnp.float32),
  )(pij.reshape(1, _N_DIM), rows, partials, tail)
  return out[0, 0]


# speedup vs baseline: 7.6351x; 7.6351x over previous
import functools

import jax
import jax.numpy as jnp
from jax import lax
from jax.experimental import pallas as pl
from jax.experimental.pallas import tpu as pltpu
from jax.experimental.pallas import tpu_sc as plsc

_N_POINTS = 1000000
_N_DIM = 16
_NC = 2
_NS = 16
_NW = _NC * _NS
_LANE = 128
_NT_FULL = _N_POINTS // _LANE
_TAIL = _N_POINTS - _NT_FULL * _LANE
_TC_BT = 84
_TC_BW = _TC_BT * _LANE
_TC_NB = 40
_SC_OFF = _TC_NB * _TC_BW
_SC_T = _NT_FULL - _TC_NB * _TC_BT
_CHT = 12
_CHW = _CHT * _LANE
_NCH = _SC_T // _CHT
_CPW = -(-_NCH // _NW)
_FULL_W = _NCH - _NW * (_CPW - 1)


def _sc_body(tt_hbm, ij_hbm, partials_hbm, rows_hbm,
             idx_v, tile_i, tile_j, rows_v, buf0, buf1, acc_v,
             sem0, sem1, gsem):
  cid = lax.axis_index("c")
  sid = lax.axis_index("s")
  wid = sid * _NC + cid

  pltpu.sync_copy(ij_hbm, idx_v)
  idx = idx_v[...]
  ii = idx[0]
  jj = idx[1]
  base_i = pl.multiple_of((ii // _LANE) * _LANE, _LANE)
  base_j = pl.multiple_of((jj // _LANE) * _LANE, _LANE)
  pltpu.sync_copy(tt_hbm.at[:, pl.ds(base_i, _LANE)], tile_i)
  pltpu.sync_copy(tt_hbm.at[:, pl.ds(base_j, _LANE)], tile_j)
  col_i = jnp.full((_N_DIM,), ii % _LANE, jnp.int32)
  col_j = jnp.full((_N_DIM,), jj % _LANE, jnp.int32)
  dim_iota = lax.iota(jnp.int32, _N_DIM)
  tis = tuple(
      plsc.load_gather(tile_i, [jnp.full((_N_DIM,), d, jnp.int32), col_i])
      for d in range(_N_DIM))
  rows_v[0, :] = plsc.load_gather(tile_i, [dim_iota, col_i])
  rows_v[1, :] = plsc.load_gather(tile_j, [dim_iota, col_j])

  bufs = (buf0, buf1)
  sems = (sem0, sem1)

  def start(c):
    g = c * _NW + wid
    if c == _CPW - 1:
      g = jnp.where(wid < _FULL_W, g, 0)
    off = pl.multiple_of(_SC_OFF + g * _CHW, _LANE)
    return pltpu.async_copy(
        tt_hbm.at[:, pl.ds(off, _CHW)], bufs[c % 2], sems[c % 2])

  def chunk_sum(buf, accs):
    @pl.loop(0, _CHW // _N_DIM, init_carry=accs)
    def accs(l, carry):
      off = l * _N_DIM
      out = []
      for d in range(_N_DIM):
        x = buf[d, pl.ds(off, _N_DIM)]
        dd = x - tis[d]
        out.append(carry[d] + jnp.exp(-(dd * dd)))
      return tuple(out)
    return accs

  zeros = tuple(jnp.zeros((_N_DIM,), jnp.float32) for _ in range(_N_DIM))
  inflight = start(0)
  accs = zeros
  for c in range(_CPW - 1):
    inflight.wait()
    nxt = start(c + 1)
    accs = chunk_sum(bufs[c % 2], accs)
    inflight = nxt
  inflight.wait()
  extra = chunk_sum(bufs[(_CPW - 1) % 2], zeros)

  zero_v = jnp.zeros((_N_DIM,), jnp.float32)
  for d in range(_N_DIM):
    acc_v[d, :] = accs[d] + jnp.where(wid < _FULL_W, extra[d], zero_v)
  pltpu.sync_copy(acc_v, partials_hbm.at[wid])

  @pl.when(wid == 0)
  def _():
    pltpu.sync_copy(rows_v, rows_hbm)


@functools.partial(
    pl.kernel,
    out_type=(
        jax.ShapeDtypeStruct((_NW, _N_DIM, _N_DIM), jnp.float32),
        jax.ShapeDtypeStruct((2, _N_DIM), jnp.float32),
    ),
    mesh=plsc.VectorSubcoreMesh(core_axis_name="c", subcore_axis_name="s"),
    compiler_params=pltpu.CompilerParams(
        use_tc_tiling_on_sc=True, needs_layout_passes=False),
    scratch_types=(
        pltpu.VMEM((_N_DIM,), jnp.int32),
        pltpu.VMEM((_N_DIM, _LANE), jnp.float32),
        pltpu.VMEM((_N_DIM, _LANE), jnp.float32),
        pltpu.VMEM((2, _N_DIM), jnp.float32),
        pltpu.VMEM((_N_DIM, _CHW), jnp.float32),
        pltpu.VMEM((_N_DIM, _CHW), jnp.float32),
        pltpu.VMEM((_N_DIM, _N_DIM), jnp.float32),
        pltpu.SemaphoreType.DMA,
        pltpu.SemaphoreType.DMA,
        pltpu.SemaphoreType.DMA,
    ),
)
def _sc_reduce(tt_hbm, ij_hbm, partials_hbm, rows_hbm, *scratch):
  _sc_body(tt_hbm, ij_hbm, partials_hbm, rows_hbm, *scratch)


def _tc_stream(ii_ref, blk_ref, tt_ref, out_ref, tile_v, ti_v, sem):
  g = pl.program_id(0)

  @pl.when(g == 0)
  def _():
    ii = ii_ref[0]
    base = pl.multiple_of((ii // _LANE) * _LANE, _LANE)
    cp = pltpu.make_async_copy(tt_ref.at[:, pl.ds(base, _LANE)], tile_v, sem)
    cp.start()
    cp.wait()
    lane = lax.broadcasted_iota(jnp.int32, (_N_DIM, _LANE), 1)
    sel = jnp.where(lane == ii % _LANE, tile_v[...], jnp.float32(0))
    ti_v[...] = jnp.sum(sel, axis=1, keepdims=True)
    out_ref[...] = jnp.zeros((1, 1), jnp.float32)

  x = blk_ref[...]
  dd = x - ti_v[...]
  out_ref[...] = out_ref[...] + jnp.sum(jnp.exp(-(dd * dd))).reshape(1, 1)


def _tc_epilogue(pij_ref, rows_ref, partials_ref, tail_ref, qtc_ref, out_ref):
  ti = rows_ref[0:1, :]
  tj = rows_ref[1:2, :]
  q_sc = jnp.sum(partials_ref[...])
  dt = tail_ref[...] - ti
  q_tail = jnp.sum(jnp.exp(-(dt * dt)))
  q_sum = q_sc + q_tail + qtc_ref[0, 0] - jnp.float32(_N_DIM)
  d = ti - tj
  p = pij_ref[...]
  t = p * (jnp.log(p) + d * d + jnp.log(q_sum))
  out_ref[...] = jnp.sum(t).reshape(1, 1)


def kernel(pij, i, j, table):
  ij = jnp.concatenate(
      [i.astype(jnp.int32), j.astype(jnp.int32),
       jnp.zeros((_N_DIM - 2,), jnp.int32)])
  tt = table.T
  partials, rows = _sc_reduce(tt, ij)
  q_tc = pl.pallas_call(
      _tc_stream,
      grid=(_TC_NB,),
      in_specs=[
          pl.BlockSpec(memory_space=pltpu.SMEM),
          pl.BlockSpec((_N_DIM, _TC_BW), lambda g: (0, g)),
          pl.BlockSpec(memory_space=pl.ANY),
      ],
      out_specs=pl.BlockSpec((1, 1), lambda g: (0, 0)),
      out_shape=jax.ShapeDtypeStruct((1, 1), jnp.float32),
      scratch_shapes=[
          pltpu.VMEM((_N_DIM, _LANE), jnp.float32),
          pltpu.VMEM((_N_DIM, 1), jnp.float32),
          pltpu.SemaphoreType.DMA,
      ],
  )(i.astype(jnp.int32), tt, tt)
  tail = lax.slice(table, (_NT_FULL * _LANE, 0), (_N_POINTS, _N_DIM))
  out = pl.pallas_call(
      _tc_epilogue,
      out_shape=jax.ShapeDtypeStruct((1, 1), jnp.float32),
  )(pij.reshape(1, _N_DIM), rows, partials, tail, q_tc)
  return out[0, 0]
